# baseline (speedup 1.0000x reference)
import functools

import jax
import jax.numpy as jnp
from jax import lax
from jax.experimental import pallas as pl
from jax.experimental.pallas import tpu as pltpu
from jax.experimental.pallas import tpu_sc as plsc

_K = 10
_L = 20
_D = _L * _K
_LANES = 16
_NC = 2
_NS = 16
_NW = _NC * _NS
_CT = 128
_NGROUPS = _CT // _LANES


def _compute_tile(t1, t2):
    for gpair in range(_NGROUPS // 2):
        sls = [pl.ds((2 * gpair + h) * _LANES, _LANES) for h in range(2)]

        def step(i, bds):
            nbds = []
            for h in range(2):
                sl = sls[h]
                bd0, bd1 = bds[h]
                xs = [t1[x * _L + i, sl] for x in range(_K)]
                ys = [t2[y * _L + i, sl] for y in range(_K)]
                m0 = []
                for v in range(_K):
                    acc = xs[0] * ys[(0 - v) % _K]
                    for x in range(1, _K):
                        acc = acc + xs[x] * ys[(x - v) % _K]
                    m0.append(acc)
                tot = m0[0]
                for v in range(1, _K):
                    tot = tot + m0[v]
                r0v = m0[0]
                p = xs[0]
                nmass = p * ys[1]
                for y in range(2, _K):
                    p = p + xs[y - 1]
                    nmass = nmass + p * ys[y]
                for v in range(_K):
                    t1[v * _L + i, sl] = bd0 * m0[v] + bd1 * m0[(v + 1) % _K]
                tmn = tot - nmass
                nb0 = bd0 * tmn + bd1 * (tmn - r0v)
                nb1 = bd0 * nmass + bd1 * (nmass + r0v)
                nbds.append((nb0, nb1))
            return tuple(nbds)

        lax.fori_loop(
            0, _L, step,
            tuple(
                (jnp.ones((16,), jnp.float32), jnp.zeros((16,), jnp.float32))
                for _ in range(2)
            ),
        )


def _sc_run(b_rows, op1_ref, op2_ref, out_ref,
            t1a, t2a, t1b, t2b, sin_a, sin_b, sout_a, sout_b):
    cols_per_w = b_rows // _NW
    ntiles = cols_per_w // _CT
    wid = lax.axis_index("s") * _NC + lax.axis_index("c")
    base = wid * cols_per_w
    bufs = ((t1a, t2a, sin_a, sout_a), (t1b, t2b, sin_b, sout_b))

    def start_in(tix, t1, t2, sem):
        c0 = base + tix * _CT
        pltpu.make_async_copy(op1_ref.at[:, pl.ds(c0, _CT)], t1, sem).start()
        pltpu.make_async_copy(op2_ref.at[:, pl.ds(c0, _CT)], t2, sem).start()

    def wait_in(t1, t2, sem):
        pltpu.make_async_copy(op1_ref.at[:, pl.ds(0, _CT)], t1, sem).wait()
        pltpu.make_async_copy(op2_ref.at[:, pl.ds(0, _CT)], t2, sem).wait()

    start_in(0, t1a, t2a, sin_a)

    def pair(p, carry):
        for par in range(2):
            t1, t2, sin, sout = bufs[par]
            n1, n2, nsin, nsout = bufs[1 - par]
            b = p * 2 + par
            wait_in(t1, t2, sin)

            @pl.when(b >= 1)
            def _():
                cprev = base + (b - 1) * _CT
                pltpu.make_async_copy(
                    n1, out_ref.at[:, pl.ds(cprev, _CT)], nsout).wait()

            @pl.when(b + 1 < ntiles)
            def _():
                c1 = base + (b + 1) * _CT
                pltpu.make_async_copy(op1_ref.at[:, pl.ds(c1, _CT)], n1, nsin).start()
                pltpu.make_async_copy(op2_ref.at[:, pl.ds(c1, _CT)], n2, nsin).start()

            _compute_tile(t1, t2)
            c0 = base + b * _CT
            pltpu.make_async_copy(t1, out_ref.at[:, pl.ds(c0, _CT)], sout).start()
        return carry

    lax.fori_loop(0, ntiles // 2, pair, 0)
    cf = base + (ntiles - 1) * _CT
    pltpu.make_async_copy(bufs[1][0], out_ref.at[:, pl.ds(cf, _CT)], bufs[1][3]).wait()


def kernel(op1, op2, sub_table, borrow_table):
    b_rows = op1.shape[0]
    op1t = jnp.transpose(op1, (2, 1, 0)).reshape(_D, b_rows)
    op2t = jnp.transpose(op2, (2, 1, 0)).reshape(_D, b_rows)
    mesh = plsc.VectorSubcoreMesh(
        core_axis_name="c", subcore_axis_name="s", num_cores=_NC, num_subcores=_NS
    )
    run = pl.kernel(
        functools.partial(_sc_run, b_rows),
        out_type=jax.ShapeDtypeStruct((_D, b_rows), jnp.float32),
        mesh=mesh,
        scratch_types=[
            pltpu.VMEM((_D, _CT), jnp.float32),
            pltpu.VMEM((_D, _CT), jnp.float32),
            pltpu.VMEM((_D, _CT), jnp.float32),
            pltpu.VMEM((_D, _CT), jnp.float32),
            pltpu.SemaphoreType.DMA,
            pltpu.SemaphoreType.DMA,
            pltpu.SemaphoreType.DMA,
            pltpu.SemaphoreType.DMA,
        ],
        compiler_params=pltpu.CompilerParams(
            use_tc_tiling_on_sc=False, needs_layout_passes=False
        ),
    )
    out = run(op1t, op2t)
    return jnp.transpose(out.reshape(_K, _L, b_rows), (2, 1, 0))


# speedup vs baseline: 1.0596x; 1.0596x over previous
import functools

import jax
import jax.numpy as jnp
from jax import lax
from jax.experimental import pallas as pl
from jax.experimental.pallas import tpu as pltpu
from jax.experimental.pallas import tpu_sc as plsc

_K = 10
_L = 20
_D = _L * _K
_LANES = 16
_NC = 2
_NS = 16
_NW = _NC * _NS
_CT = 64
_NGROUPS = _CT // _LANES


def _compute_tile(t1, t2, to):
    for gpair in range(_NGROUPS // 2):
        sls = [pl.ds((2 * gpair + h) * _LANES, _LANES) for h in range(2)]

        def step(i, bds):
            nbds = []
            for h in range(2):
                sl = sls[h]
                bd0, bd1 = bds[h]
                xs = [t1[x * _L + i, sl] for x in range(_K)]
                ys = [t2[y * _L + i, sl] for y in range(_K)]
                m0 = []
                for v in range(_K):
                    acc = xs[0] * ys[(0 - v) % _K]
                    for x in range(1, _K):
                        acc = acc + xs[x] * ys[(x - v) % _K]
                    m0.append(acc)
                tot = m0[0]
                for v in range(1, _K):
                    tot = tot + m0[v]
                r0v = m0[0]
                p = xs[0]
                nmass = p * ys[1]
                for y in range(2, _K):
                    p = p + xs[y - 1]
                    nmass = nmass + p * ys[y]
                for v in range(_K):
                    to[v * _L + i, sl] = bd0 * m0[v] + bd1 * m0[(v + 1) % _K]
                tmn = tot - nmass
                nb0 = bd0 * tmn + bd1 * (tmn - r0v)
                nb1 = bd0 * nmass + bd1 * (nmass + r0v)
                nbds.append((nb0, nb1))
            return tuple(nbds)

        lax.fori_loop(
            0, _L, step,
            tuple(
                (jnp.ones((16,), jnp.float32), jnp.zeros((16,), jnp.float32))
                for _ in range(2)
            ),
        )


def _sc_run(b_rows, op1_ref, op2_ref, out_ref,
            t1a, t2a, toa, t1b, t2b, tob, sin_a, sin_b, sout_a, sout_b):
    cols_per_w = b_rows // _NW
    ntiles = cols_per_w // _CT
    wid = lax.axis_index("s") * _NC + lax.axis_index("c")
    base = wid * cols_per_w
    bufs = ((t1a, t2a, toa, sin_a, sout_a), (t1b, t2b, tob, sin_b, sout_b))

    def start_in(tix, t1, t2, sem):
        c0 = base + tix * _CT
        pltpu.make_async_copy(op1_ref.at[:, pl.ds(c0, _CT)], t1, sem).start()
        pltpu.make_async_copy(op2_ref.at[:, pl.ds(c0, _CT)], t2, sem).start()

    def wait_in(t1, t2, sem):
        pltpu.make_async_copy(op1_ref.at[:, pl.ds(0, _CT)], t1, sem).wait()
        pltpu.make_async_copy(op2_ref.at[:, pl.ds(0, _CT)], t2, sem).wait()

    start_in(0, t1a, t2a, sin_a)

    def pair(p, carry):
        for par in range(2):
            t1, t2, to, sin, sout = bufs[par]
            n1, n2, _, nsin, _ = bufs[1 - par]
            b = p * 2 + par
            wait_in(t1, t2, sin)

            @pl.when(b + 1 < ntiles)
            def _():
                c1 = base + (b + 1) * _CT
                pltpu.make_async_copy(op1_ref.at[:, pl.ds(c1, _CT)], n1, nsin).start()
                pltpu.make_async_copy(op2_ref.at[:, pl.ds(c1, _CT)], n2, nsin).start()

            @pl.when(b >= 2)
            def _():
                c2 = base + (b - 2) * _CT
                pltpu.make_async_copy(to, out_ref.at[:, pl.ds(c2, _CT)], sout).wait()

            _compute_tile(t1, t2, to)
            c0 = base + b * _CT
            pltpu.make_async_copy(to, out_ref.at[:, pl.ds(c0, _CT)], sout).start()
        return carry

    lax.fori_loop(0, ntiles // 2, pair, 0)
    ce = base + (ntiles - 2) * _CT
    pltpu.make_async_copy(toa, out_ref.at[:, pl.ds(ce, _CT)], sout_a).wait()
    cf = base + (ntiles - 1) * _CT
    pltpu.make_async_copy(tob, out_ref.at[:, pl.ds(cf, _CT)], sout_b).wait()


def kernel(op1, op2, sub_table, borrow_table):
    b_rows = op1.shape[0]
    op1t = jnp.transpose(op1, (2, 1, 0)).reshape(_D, b_rows)
    op2t = jnp.transpose(op2, (2, 1, 0)).reshape(_D, b_rows)
    mesh = plsc.VectorSubcoreMesh(
        core_axis_name="c", subcore_axis_name="s", num_cores=_NC, num_subcores=_NS
    )
    run = pl.kernel(
        functools.partial(_sc_run, b_rows),
        out_type=jax.ShapeDtypeStruct((_D, b_rows), jnp.float32),
        mesh=mesh,
        scratch_types=[
            pltpu.VMEM((_D, _CT), jnp.float32),
            pltpu.VMEM((_D, _CT), jnp.float32),
            pltpu.VMEM((_D, _CT), jnp.float32),
            pltpu.VMEM((_D, _CT), jnp.float32),
            pltpu.VMEM((_D, _CT), jnp.float32),
            pltpu.VMEM((_D, _CT), jnp.float32),
            pltpu.SemaphoreType.DMA,
            pltpu.SemaphoreType.DMA,
            pltpu.SemaphoreType.DMA,
            pltpu.SemaphoreType.DMA,
        ],
        compiler_params=pltpu.CompilerParams(
            use_tc_tiling_on_sc=False, needs_layout_passes=False
        ),
    )
    out = run(op1t, op2t)
    return jnp.transpose(out.reshape(_K, _L, b_rows), (2, 1, 0))
